# Initial kernel scaffold; baseline (speedup 1.0000x reference)
import jax
import jax.numpy as jnp
from jax.experimental import pallas as pl


def kernel(x, spin, params):
    raise NotImplementedError("write your pallas kernel here")


# speedup vs baseline: 1.5651x; 1.5651x over previous
import numpy as np
import jax
import jax.numpy as jnp
from jax.experimental import pallas as pl

N_PART = 16
DIM = 3
NODE_H = 32
EDGE_H = 32
N_STEPS = 2
E_DENSE = N_PART * N_PART
N_EDGE = N_PART * (N_PART - 1)
WB = 64


def _edge_slot_index():
    idx = []
    for i in range(N_PART):
        for j in range(N_PART):
            if i != j:
                idx.append(i * N_PART + j)
    return np.asarray(idx, np.int32)


_EDGE_IDX = _edge_slot_index()


def _gelu(x):
    return 0.5 * x * (1.0 + jax.lax.erf(x * np.float32(1.0 / np.sqrt(2.0))))


def _fwd_body(xT, spT, node_Wt, node_b, ee_W1t, ee_b1, ee_W2t, ee_b2,
              v2e_Wt, euA, euB, euC, eu_b1, eu_W2t, eu_b2, e2v_Wt,
              nuH, nuA, nu_b1, nu_W2t, nu_b2,
              W0v, W0e, w_rp, w_r2, fb0, fW1t, fb1, fW2t, fb2, out_ref):
    f32 = jnp.float32
    x = xT[...]
    sp = spT[...]
    NB = N_PART * WB
    EB = E_DENSE * WB

    nw = node_Wt[...]
    h_v = (x.reshape(NB, DIM) @ nw[:DIM]
           + sp.reshape(NB, 1) * nw[DIM:DIM + 1]
           + node_b[...])

    xi = jnp.broadcast_to(x[:, None], (N_PART, N_PART, WB, DIM)).reshape(EB, DIM)
    xj = jnp.broadcast_to(x[None, :], (N_PART, N_PART, WB, DIM)).reshape(EB, DIM)
    dr = xj - xi
    r2 = jnp.sum(dr * dr, axis=-1, keepdims=True)
    rr = jnp.sqrt(r2 + 1e-12)
    w1 = ee_W1t[...]
    t = dr @ w1[:DIM] + rr * w1[DIM:DIM + 1] + r2 * w1[DIM + 1:DIM + 2] + ee_b1[...]
    h_e = _gelu(t) @ ee_W2t[...] + ee_b2[...]

    ii = jax.lax.broadcasted_iota(jnp.int32, (N_PART, N_PART, 1, 1), 0)
    jj = jax.lax.broadcasted_iota(jnp.int32, (N_PART, N_PART, 1, 1), 1)
    mask = (ii != jj).astype(f32)

    for s in range(N_STEPS):
        v2e = h_v @ v2e_Wt[s]
        s_src = v2e @ euB[s]
        s_dst = v2e @ euC[s]
        bs = jnp.broadcast_to(
            s_src.reshape(N_PART, 1, WB, EDGE_H),
            (N_PART, N_PART, WB, EDGE_H)).reshape(EB, EDGE_H)
        bd = jnp.broadcast_to(
            s_dst.reshape(1, N_PART, WB, EDGE_H),
            (N_PART, N_PART, WB, EDGE_H)).reshape(EB, EDGE_H)
        t = h_e @ euA[s] + bs + bd + eu_b1[s]
        h_e = _gelu(t) @ eu_W2t[s] + eu_b2[s]
        msg = h_e @ e2v_Wt[s]
        m4 = msg.reshape(N_PART, N_PART, WB, NODE_H) * mask
        agg = m4.sum(axis=0).reshape(NB, NODE_H) * (1.0 / (N_PART - 1))
        t2 = h_v @ nuH[s] + agg @ nuA[s] + nu_b1[s]
        h_v = _gelu(t2) @ nu_W2t[s] + nu_b2[s]

    hv4 = h_v.reshape(N_PART, WB, NODE_H)
    acc = jnp.zeros((WB, NODE_H), f32) + fb0[...]
    for i in range(N_PART):
        acc = acc + hv4[i] @ W0v[i]
    he4 = h_e.reshape(E_DENSE, WB, EDGE_H)
    u = jax.lax.dot_general(he4, W0e[...], (((2,), (1,)), ((0,), (0,))),
                            preferred_element_type=f32)
    acc = acc + u.sum(axis=0)
    r2a = jnp.sum(jnp.sum(x * x, axis=0), axis=-1, keepdims=True)
    df = x[0] - x[1]
    rp = jnp.sqrt(jnp.sum(df * df, axis=-1, keepdims=True) + 1e-12)
    acc = acc + rp * w_rp[...] + r2a * w_r2[...]
    h = _gelu(acc)
    h = _gelu(h @ fW1t[...] + fb1[...])
    out_ref[...] = h @ fW2t[...] + fb2[...]


def kernel(x, spin, params):
    B = x.shape[0]
    f32 = x.dtype
    p = params

    xT = x.transpose(1, 0, 2)
    spT = spin.astype(f32).transpose(1, 0)[..., None]

    W0 = p["fh_W0"]
    nv = N_PART * NODE_H
    ne = N_EDGE * EDGE_H
    W0v = W0[:, :nv].reshape(NODE_H, N_PART, NODE_H).transpose(1, 2, 0)
    W0e_real = W0[:, nv:nv + ne].reshape(NODE_H, N_EDGE, EDGE_H).transpose(1, 2, 0)
    W0e = jnp.zeros((E_DENSE, EDGE_H, NODE_H), f32).at[_EDGE_IDX].set(W0e_real)
    w_rp = W0[:, nv + ne][None, :]
    w_r2 = W0[:, nv + ne + 1][None, :]

    eu1 = p["eu_W1"]
    nu1 = p["nu_W1"]
    weights = [
        p["node_W"].T, p["node_b"][None, :],
        p["ee_W1"].T, p["ee_b1"][None, :],
        p["ee_W2"].T, p["ee_b2"][None, :],
        p["v2e_W"].transpose(0, 2, 1),
        eu1[:, :, :EDGE_H].transpose(0, 2, 1),
        eu1[:, :, EDGE_H:2 * EDGE_H].transpose(0, 2, 1),
        eu1[:, :, 2 * EDGE_H:].transpose(0, 2, 1),
        p["eu_b1"][:, None, :],
        p["eu_W2"].transpose(0, 2, 1), p["eu_b2"][:, None, :],
        p["e2v_W"].transpose(0, 2, 1),
        nu1[:, :, :NODE_H].transpose(0, 2, 1),
        nu1[:, :, NODE_H:].transpose(0, 2, 1),
        p["nu_b1"][:, None, :],
        p["nu_W2"].transpose(0, 2, 1), p["nu_b2"][:, None, :],
        W0v, W0e, w_rp, w_r2,
        p["fh_b0"][None, :],
        p["fh_W1"].T, p["fh_b1"][None, :],
        p["fh_W2"].T, p["fh_b2"][None, :],
    ]

    grid = (B // WB,)
    in_specs = [
        pl.BlockSpec((N_PART, WB, DIM), lambda i: (0, i, 0)),
        pl.BlockSpec((N_PART, WB, 1), lambda i: (0, i, 0)),
    ] + [pl.BlockSpec(w.shape, lambda i, nd=w.ndim: (0,) * nd) for w in weights]

    out = pl.pallas_call(
        _fwd_body,
        grid=grid,
        in_specs=in_specs,
        out_specs=pl.BlockSpec((WB, 1), lambda i: (i, 0)),
        out_shape=jax.ShapeDtypeStruct((B, 1), f32),
    )(xT, spT, *weights)
    return out


# speedup vs baseline: 2.7598x; 1.7633x over previous
import numpy as np
import jax
import jax.numpy as jnp
from jax.experimental import pallas as pl

N_PART = 16
DIM = 3
NODE_H = 32
EDGE_H = 32
N_STEPS = 2
E_DENSE = N_PART * N_PART
N_EDGE = N_PART * (N_PART - 1)
PK = 4
JHI = N_PART // PK
LANES = PK * EDGE_H
WB = 64


def _edge_slot_index():
    idx = []
    for i in range(N_PART):
        for j in range(N_PART):
            if i != j:
                idx.append(i * N_PART + j)
    return np.asarray(idx, np.int32)


_EDGE_IDX = _edge_slot_index()

_S12 = np.zeros((PK * DIM, PK), np.float32)
for _l in range(PK):
    for _d in range(DIM):
        _S12[_l * DIM + _d, _l] = 1.0

_MASKP = np.ones((N_PART, JHI, 1, LANES), np.float32)
for _i in range(N_PART):
    for _jh in range(JHI):
        for _jl in range(PK):
            if _i == _jh * PK + _jl:
                _MASKP[_i, _jh, 0, _jl * EDGE_H:(_jl + 1) * EDGE_H] = 0.0


def _gelu(x):
    return 0.5 * x * (1.0 + jax.lax.erf(x * np.float32(1.0 / np.sqrt(2.0))))


def _k4(w):
    return jax.scipy.linalg.block_diag(w, w, w, w)


def _fwd_body(xT, xit, xp4, spT, node_Wt, node_b, S12,
              eeW1_K, eeW1rr_K, eeW1r2_K, ee_b1t, eeW2_K, ee_b2t,
              v2e_Wt, euB_tile, euC_pl, euA_K, eu_b1t, euW2_K, eu_b2t,
              e2v_K, maskp, nuHt, nuA_sel, nu_b1, nuW2t, nu_b2,
              W0v, W0ep, w_rp, w_r2, fb0, fW1t, fb1, fW2t, fb2, out_ref):
    f32 = jnp.float32
    x = xT[...]
    sp = spT[...]
    NB = N_PART * WB
    EB4 = N_PART * JHI * WB

    nw = node_Wt[...]
    h_v = (x.reshape(NB, DIM) @ nw[:DIM]
           + sp.reshape(NB, 1) * nw[DIM:DIM + 1]
           + node_b[...])

    xi = xit[...]
    xj = xp4[...]
    drb = (jnp.broadcast_to(xj[None], (N_PART, JHI, WB, PK * DIM))
           - jnp.broadcast_to(xi[:, None], (N_PART, JHI, WB, PK * DIM)))
    dr = drb.reshape(EB4, PK * DIM)
    r2 = (dr * dr) @ S12[...]
    rr = jnp.sqrt(r2 + 1e-12)
    t = dr @ eeW1_K[...] + rr @ eeW1rr_K[...] + r2 @ eeW1r2_K[...] + ee_b1t[...]
    h_e = _gelu(t) @ eeW2_K[...] + ee_b2t[...]

    for s in range(N_STEPS):
        v2e = h_v @ v2e_Wt[s]
        bs = (v2e @ euB_tile[s]).reshape(N_PART, 1, WB, LANES)
        v2e4 = v2e.reshape(JHI, PK, WB, NODE_H)
        sd = v2e4[:, 0].reshape(JHI * WB, NODE_H) @ euC_pl[s, 0]
        for l in range(1, PK):
            sd = sd + v2e4[:, l].reshape(JHI * WB, NODE_H) @ euC_pl[s, l]
        bd = sd.reshape(1, JHI, WB, LANES)
        t = ((h_e @ euA_K[s]).reshape(N_PART, JHI, WB, LANES)
             + bs + bd + eu_b1t[s])
        h_e = _gelu(t.reshape(EB4, LANES)) @ euW2_K[s] + eu_b2t[s]
        msg = h_e @ e2v_K[s]
        aggp = (msg.reshape(N_PART, JHI, WB, LANES) * maskp[...]).sum(axis=0)
        parts = []
        for jh in range(JHI):
            ap = aggp[jh]
            for jl in range(PK):
                parts.append(ap @ nuA_sel[s, jl])
        t2agg = jnp.concatenate(parts, axis=0)
        t2 = h_v @ nuHt[s] + t2agg + nu_b1[s]
        h_v = _gelu(t2) @ nuW2t[s] + nu_b2[s]

    hv4 = h_v.reshape(N_PART, WB, NODE_H)
    acc = jnp.zeros((WB, NODE_H), f32) + fb0[...]
    for i in range(N_PART):
        acc = acc + hv4[i] @ W0v[i]
    he4p = h_e.reshape(N_PART * JHI, WB, LANES)
    u = jax.lax.dot_general(he4p, W0ep[...], (((2,), (1,)), ((0,), (0,))),
                            preferred_element_type=f32)
    acc = acc + u.sum(axis=0)
    r2a = jnp.sum(jnp.sum(x * x, axis=0), axis=-1, keepdims=True)
    df = x[0] - x[1]
    rp = jnp.sqrt(jnp.sum(df * df, axis=-1, keepdims=True) + 1e-12)
    acc = acc + rp * w_rp[...] + r2a * w_r2[...]
    h = _gelu(acc)
    h = _gelu(h @ fW1t[...] + fb1[...])
    out_ref[...] = h @ fW2t[...] + fb2[...]


def kernel(x, spin, params):
    B = x.shape[0]
    f32 = x.dtype
    p = params

    xT = x.transpose(1, 0, 2)
    xit = jnp.tile(xT, (1, 1, PK))
    xp4 = x.reshape(B, JHI, PK * DIM).transpose(1, 0, 2)
    spT = spin.astype(f32).transpose(1, 0)[..., None]

    W0 = p["fh_W0"]
    nv = N_PART * NODE_H
    ne = N_EDGE * EDGE_H
    W0v = W0[:, :nv].reshape(NODE_H, N_PART, NODE_H).transpose(1, 2, 0)
    W0e_real = W0[:, nv:nv + ne].reshape(NODE_H, N_EDGE, EDGE_H).transpose(1, 2, 0)
    W0e = jnp.zeros((E_DENSE, EDGE_H, NODE_H), f32).at[_EDGE_IDX].set(W0e_real)
    W0ep = W0e.reshape(N_PART * JHI, LANES, NODE_H)
    w_rp = W0[:, nv + ne][None, :]
    w_r2 = W0[:, nv + ne + 1][None, :]

    ee_W1t = p["ee_W1"].T
    eeW1_K = _k4(ee_W1t[:DIM])
    eeW1rr_K = _k4(ee_W1t[DIM:DIM + 1])
    eeW1r2_K = _k4(ee_W1t[DIM + 1:DIM + 2])

    eu1 = p["eu_W1"]
    euAt = eu1[:, :, :EDGE_H].transpose(0, 2, 1)
    euBt = eu1[:, :, EDGE_H:2 * EDGE_H].transpose(0, 2, 1)
    euCt = eu1[:, :, 2 * EDGE_H:].transpose(0, 2, 1)
    euA_K = jnp.stack([_k4(euAt[s]) for s in range(N_STEPS)])
    euB_tile = jnp.tile(euBt, (1, 1, PK))
    euC_pl = jnp.stack([
        jnp.stack([jnp.pad(euCt[s], ((0, 0), (l * EDGE_H, LANES - (l + 1) * EDGE_H)))
                   for l in range(PK)])
        for s in range(N_STEPS)])
    euW2_K = jnp.stack([_k4(p["eu_W2"][s].T) for s in range(N_STEPS)])
    e2v_K = jnp.stack([_k4(p["e2v_W"][s].T) for s in range(N_STEPS)])

    nu1 = p["nu_W1"]
    nuHt = nu1[:, :, :NODE_H].transpose(0, 2, 1)
    nuAt = nu1[:, :, NODE_H:].transpose(0, 2, 1) * (1.0 / (N_PART - 1))
    nuA_sel = jnp.stack([
        jnp.stack([jnp.pad(nuAt[s], ((l * NODE_H, LANES - (l + 1) * NODE_H), (0, 0)))
                   for l in range(PK)])
        for s in range(N_STEPS)])

    tile4 = lambda b: jnp.tile(b, PK)[None, :]
    weights = [
        p["node_W"].T, p["node_b"][None, :],
        jnp.asarray(_S12),
        eeW1_K, eeW1rr_K, eeW1r2_K,
        tile4(p["ee_b1"]), _k4(p["ee_W2"].T), tile4(p["ee_b2"]),
        p["v2e_W"].transpose(0, 2, 1),
        euB_tile, euC_pl, euA_K,
        jnp.tile(p["eu_b1"], (1, PK))[:, None, :],
        euW2_K,
        jnp.tile(p["eu_b2"], (1, PK))[:, None, :],
        e2v_K,
        jnp.asarray(_MASKP),
        nuHt, nuA_sel,
        p["nu_b1"][:, None, :],
        p["nu_W2"].transpose(0, 2, 1), p["nu_b2"][:, None, :],
        W0v, W0ep, w_rp, w_r2,
        p["fh_b0"][None, :],
        p["fh_W1"].T, p["fh_b1"][None, :],
        p["fh_W2"].T, p["fh_b2"][None, :],
    ]

    grid = (B // WB,)
    in_specs = [
        pl.BlockSpec((N_PART, WB, DIM), lambda i: (0, i, 0)),
        pl.BlockSpec((N_PART, WB, PK * DIM), lambda i: (0, i, 0)),
        pl.BlockSpec((JHI, WB, PK * DIM), lambda i: (0, i, 0)),
        pl.BlockSpec((N_PART, WB, 1), lambda i: (0, i, 0)),
    ] + [pl.BlockSpec(w.shape, lambda i, nd=w.ndim: (0,) * nd) for w in weights]

    out = pl.pallas_call(
        _fwd_body,
        grid=grid,
        in_specs=in_specs,
        out_specs=pl.BlockSpec((WB, 1), lambda i: (i, 0)),
        out_shape=jax.ShapeDtypeStruct((B, 1), f32),
    )(xT, xit, xp4, spT, *weights)
    return out


# speedup vs baseline: 3.0166x; 1.0930x over previous
import numpy as np
import jax
import jax.numpy as jnp
from jax.experimental import pallas as pl

N_PART = 16
DIM = 3
NODE_H = 32
EDGE_H = 32
N_STEPS = 2
E_DENSE = N_PART * N_PART
N_EDGE = N_PART * (N_PART - 1)
PK = 4
JHI = N_PART // PK
LANES = PK * EDGE_H
WB = 128


def _edge_slot_index():
    idx = []
    for i in range(N_PART):
        for j in range(N_PART):
            if i != j:
                idx.append(i * N_PART + j)
    return np.asarray(idx, np.int32)


_EDGE_IDX = _edge_slot_index()

_S12 = np.zeros((PK * DIM, PK), np.float32)
for _l in range(PK):
    for _d in range(DIM):
        _S12[_l * DIM + _d, _l] = 1.0

_MASKP = np.ones((N_PART, JHI, 1, LANES), np.float32)
for _i in range(N_PART):
    for _jh in range(JHI):
        for _jl in range(PK):
            if _i == _jh * PK + _jl:
                _MASKP[_i, _jh, 0, _jl * EDGE_H:(_jl + 1) * EDGE_H] = 0.0


def _gelu(x):
    return 0.5 * x * (1.0 + jax.lax.erf(x * np.float32(1.0 / np.sqrt(2.0))))


def _k4(w):
    return jax.scipy.linalg.block_diag(w, w, w, w)


def _fwd_body(xT, xit, xp4, spT, node_Wt, node_b, S12,
              eeW1_K, eeW1rr_K, eeW1r2_K, ee_b1t, eeW2_K, ee_b2t,
              v2e_Wt, euB_tile, euC_pl, euA_K, eu_b1t, euW2_K, eu_b2t,
              e2v_K, maskp, nuHt, nuA_sel, nu_b1, nuW2t, nu_b2,
              W0v, W0ep, w_rp, w_r2, fb0, fW1t, fb1, fW2t, fb2, out_ref):
    f32 = jnp.float32
    x = xT[...]
    sp = spT[...]
    NB = N_PART * WB
    EB4 = N_PART * JHI * WB

    nw = node_Wt[...]
    h_v = (x.reshape(NB, DIM) @ nw[:DIM]
           + sp.reshape(NB, 1) * nw[DIM:DIM + 1]
           + node_b[...])

    xi = xit[...]
    xj = xp4[...]
    drb = (jnp.broadcast_to(xj[None], (N_PART, JHI, WB, PK * DIM))
           - jnp.broadcast_to(xi[:, None], (N_PART, JHI, WB, PK * DIM)))
    dr = drb.reshape(EB4, PK * DIM)
    r2 = (dr * dr) @ S12[...]
    rr = jnp.sqrt(r2 + 1e-12)
    t = dr @ eeW1_K[...] + rr @ eeW1rr_K[...] + r2 @ eeW1r2_K[...] + ee_b1t[...]
    h_e = _gelu(t) @ eeW2_K[...] + ee_b2t[...]

    for s in range(N_STEPS):
        v2e = h_v @ v2e_Wt[s]
        bs = (v2e @ euB_tile[s]).reshape(N_PART, 1, WB, LANES)
        v2e4 = v2e.reshape(JHI, PK, WB, NODE_H)
        sd = v2e4[:, 0].reshape(JHI * WB, NODE_H) @ euC_pl[s, 0]
        for l in range(1, PK):
            sd = sd + v2e4[:, l].reshape(JHI * WB, NODE_H) @ euC_pl[s, l]
        bd = sd.reshape(1, JHI, WB, LANES)
        t = ((h_e @ euA_K[s]).reshape(N_PART, JHI, WB, LANES)
             + bs + bd + eu_b1t[s])
        h_e = _gelu(t.reshape(EB4, LANES)) @ euW2_K[s] + eu_b2t[s]
        msg = h_e @ e2v_K[s]
        aggp = (msg.reshape(N_PART, JHI, WB, LANES) * maskp[...]).sum(axis=0)
        parts = []
        for jh in range(JHI):
            ap = aggp[jh]
            for jl in range(PK):
                parts.append(ap @ nuA_sel[s, jl])
        t2agg = jnp.concatenate(parts, axis=0)
        t2 = h_v @ nuHt[s] + t2agg + nu_b1[s]
        h_v = _gelu(t2) @ nuW2t[s] + nu_b2[s]

    hv4 = h_v.reshape(N_PART, WB, NODE_H)
    acc = jnp.zeros((WB, NODE_H), f32) + fb0[...]
    for i in range(N_PART):
        acc = acc + hv4[i] @ W0v[i]
    he4p = h_e.reshape(N_PART * JHI, WB, LANES)
    u = jax.lax.dot_general(he4p, W0ep[...], (((2,), (1,)), ((0,), (0,))),
                            preferred_element_type=f32)
    acc = acc + u.sum(axis=0)
    r2a = jnp.sum(jnp.sum(x * x, axis=0), axis=-1, keepdims=True)
    df = x[0] - x[1]
    rp = jnp.sqrt(jnp.sum(df * df, axis=-1, keepdims=True) + 1e-12)
    acc = acc + rp * w_rp[...] + r2a * w_r2[...]
    h = _gelu(acc)
    h = _gelu(h @ fW1t[...] + fb1[...])
    out_ref[...] = h @ fW2t[...] + fb2[...]


def kernel(x, spin, params):
    B = x.shape[0]
    f32 = x.dtype
    p = params

    xT = x.transpose(1, 0, 2)
    xit = jnp.tile(xT, (1, 1, PK))
    xp4 = x.reshape(B, JHI, PK * DIM).transpose(1, 0, 2)
    spT = spin.astype(f32).transpose(1, 0)[..., None]

    W0 = p["fh_W0"]
    nv = N_PART * NODE_H
    ne = N_EDGE * EDGE_H
    W0v = W0[:, :nv].reshape(NODE_H, N_PART, NODE_H).transpose(1, 2, 0)
    W0e_real = W0[:, nv:nv + ne].reshape(NODE_H, N_EDGE, EDGE_H).transpose(1, 2, 0)
    W0e = jnp.zeros((E_DENSE, EDGE_H, NODE_H), f32).at[_EDGE_IDX].set(W0e_real)
    W0ep = W0e.reshape(N_PART * JHI, LANES, NODE_H)
    w_rp = W0[:, nv + ne][None, :]
    w_r2 = W0[:, nv + ne + 1][None, :]

    ee_W1t = p["ee_W1"].T
    eeW1_K = _k4(ee_W1t[:DIM])
    eeW1rr_K = _k4(ee_W1t[DIM:DIM + 1])
    eeW1r2_K = _k4(ee_W1t[DIM + 1:DIM + 2])

    eu1 = p["eu_W1"]
    euAt = eu1[:, :, :EDGE_H].transpose(0, 2, 1)
    euBt = eu1[:, :, EDGE_H:2 * EDGE_H].transpose(0, 2, 1)
    euCt = eu1[:, :, 2 * EDGE_H:].transpose(0, 2, 1)
    euA_K = jnp.stack([_k4(euAt[s]) for s in range(N_STEPS)])
    euB_tile = jnp.tile(euBt, (1, 1, PK))
    euC_pl = jnp.stack([
        jnp.stack([jnp.pad(euCt[s], ((0, 0), (l * EDGE_H, LANES - (l + 1) * EDGE_H)))
                   for l in range(PK)])
        for s in range(N_STEPS)])
    euW2_K = jnp.stack([_k4(p["eu_W2"][s].T) for s in range(N_STEPS)])
    e2v_K = jnp.stack([_k4(p["e2v_W"][s].T) for s in range(N_STEPS)])

    nu1 = p["nu_W1"]
    nuHt = nu1[:, :, :NODE_H].transpose(0, 2, 1)
    nuAt = nu1[:, :, NODE_H:].transpose(0, 2, 1) * (1.0 / (N_PART - 1))
    nuA_sel = jnp.stack([
        jnp.stack([jnp.pad(nuAt[s], ((l * NODE_H, LANES - (l + 1) * NODE_H), (0, 0)))
                   for l in range(PK)])
        for s in range(N_STEPS)])

    tile4 = lambda b: jnp.tile(b, PK)[None, :]
    weights = [
        p["node_W"].T, p["node_b"][None, :],
        jnp.asarray(_S12),
        eeW1_K, eeW1rr_K, eeW1r2_K,
        tile4(p["ee_b1"]), _k4(p["ee_W2"].T), tile4(p["ee_b2"]),
        p["v2e_W"].transpose(0, 2, 1),
        euB_tile, euC_pl, euA_K,
        jnp.tile(p["eu_b1"], (1, PK))[:, None, :],
        euW2_K,
        jnp.tile(p["eu_b2"], (1, PK))[:, None, :],
        e2v_K,
        jnp.asarray(_MASKP),
        nuHt, nuA_sel,
        p["nu_b1"][:, None, :],
        p["nu_W2"].transpose(0, 2, 1), p["nu_b2"][:, None, :],
        W0v, W0ep, w_rp, w_r2,
        p["fh_b0"][None, :],
        p["fh_W1"].T, p["fh_b1"][None, :],
        p["fh_W2"].T, p["fh_b2"][None, :],
    ]

    grid = (B // WB,)
    in_specs = [
        pl.BlockSpec((N_PART, WB, DIM), lambda i: (0, i, 0)),
        pl.BlockSpec((N_PART, WB, PK * DIM), lambda i: (0, i, 0)),
        pl.BlockSpec((JHI, WB, PK * DIM), lambda i: (0, i, 0)),
        pl.BlockSpec((N_PART, WB, 1), lambda i: (0, i, 0)),
    ] + [pl.BlockSpec(w.shape, lambda i, nd=w.ndim: (0,) * nd) for w in weights]

    out = pl.pallas_call(
        _fwd_body,
        grid=grid,
        in_specs=in_specs,
        out_specs=pl.BlockSpec((WB, 1), lambda i: (i, 0)),
        out_shape=jax.ShapeDtypeStruct((B, 1), f32),
    )(xT, xit, xp4, spT, *weights)
    return out
